# Initial kernel scaffold; baseline (speedup 1.0000x reference)
import functools

import jax
import jax.numpy as jnp
from jax import lax
from jax.experimental import pallas as pl
from jax.experimental.pallas import tpu as pltpu
from jax.experimental.pallas import tpu_sc as plsc

D = 128
K = 32
OUT = 384
NC = 2
NS = 16
NW = NC * NS
BP = 10240
CB = BP // NW
C = 8
NCHUNK = CB // C
CK = C * K


def _sc_gather_mean(nodes_p, adj2d, dis2d, feat_table):
    mesh = plsc.VectorSubcoreMesh(core_axis_name="c", subcore_axis_name="s")

    @functools.partial(
        pl.kernel,
        out_type=(
            jax.ShapeDtypeStruct((BP, D), jnp.float32),
            jax.ShapeDtypeStruct((BP, D), jnp.float32),
            jax.ShapeDtypeStruct((BP, D), jnp.float32),
        ),
        mesh=mesh,
        scratch_types=[
            pltpu.VMEM((C,), jnp.int32),
            pltpu.VMEM((2, 128), jnp.int32),
            pltpu.VMEM((2, 128), jnp.int32),
            pltpu.VMEM((C, D), jnp.float32),
            pltpu.VMEM((CK, D), jnp.float32),
            pltpu.VMEM((CK, D), jnp.float32),
            pltpu.VMEM((C, D), jnp.float32),
            pltpu.VMEM((C, D), jnp.float32),
            pltpu.SemaphoreType.DMA,
        ],
    )
    def sc_kernel(nodes_hbm, adj_hbm, dis_hbm, table_hbm,
                  self_out, adj_out, dis_out,
                  nidx, aidx, didx, srows, arows, drows, amean, dmean, sem):
        wid = lax.axis_index("s") * NC + lax.axis_index("c")

        def chunk_body(c, carry):
            base = wid * CB + c * C
            rbase = wid * (CB * K // 128) + c * (C * K // 128)

            pltpu.sync_copy(nodes_hbm.at[pl.ds(base, C)], nidx)
            pltpu.sync_copy(adj_hbm.at[pl.ds(rbase, 2)], aidx)
            pltpu.sync_copy(dis_hbm.at[pl.ds(rbase, 2)], didx)

            cps = [
                pltpu.async_copy(table_hbm.at[nidx], srows, sem),
                pltpu.async_copy(table_hbm.at[aidx.at[0]],
                                 arows.at[pl.ds(0, 128)], sem),
                pltpu.async_copy(table_hbm.at[aidx.at[1]],
                                 arows.at[pl.ds(128, 128)], sem),
                pltpu.async_copy(table_hbm.at[didx.at[0]],
                                 drows.at[pl.ds(0, 128)], sem),
                pltpu.async_copy(table_hbm.at[didx.at[1]],
                                 drows.at[pl.ds(128, 128)], sem),
            ]
            for cp in cps:
                cp.wait()

            def node_body(i, carry2):
                rb = i * K
                for d in range(D // 16):
                    sl = pl.ds(d * 16, 16)

                    def kacc(k4, accs):
                        a, dd = accs
                        for u in range(4):
                            r = rb + k4 * 4 + u
                            a = a + arows[r, sl]
                            dd = dd + drows[r, sl]
                        return (a, dd)

                    zero = jnp.zeros((16,), jnp.float32)
                    a, dd = lax.fori_loop(0, K // 4, kacc, (zero, zero))
                    amean[i, sl] = a * (1.0 / K)
                    dmean[i, sl] = dd * (1.0 / K)
                return carry2

            lax.fori_loop(0, C, node_body, 0)

            pltpu.sync_copy(srows, self_out.at[pl.ds(base, C)])
            pltpu.sync_copy(amean, adj_out.at[pl.ds(base, C)])
            pltpu.sync_copy(dmean, dis_out.at[pl.ds(base, C)])
            return carry

        lax.fori_loop(0, NCHUNK, chunk_body, 0)

    return sc_kernel(nodes_p, adj2d, dis2d, feat_table)


def _tc_finish(selfs, adjm, dism, wt_self, wt_adj, wt_dis, bias2d):
    BM = 512

    def body(s_ref, a_ref, d_ref, ws_ref, wa_ref, wd_ref, b_ref, o_ref):
        hs = jnp.dot(s_ref[...], ws_ref[...], preferred_element_type=jnp.float32)
        ha = jnp.dot(a_ref[...], wa_ref[...], preferred_element_type=jnp.float32)
        hd = jnp.dot(d_ref[...], wd_ref[...], preferred_element_type=jnp.float32)
        h = jnp.concatenate([hs, ha, hd], axis=-1) + b_ref[...]
        h = jnp.where(h >= 0, h, 0.2 * h)
        n = jnp.sqrt(jnp.sum(h * h, axis=-1, keepdims=True))
        o_ref[...] = h / jnp.maximum(n, 1e-12)

    return pl.pallas_call(
        body,
        grid=(BP // BM,),
        in_specs=[
            pl.BlockSpec((BM, D), lambda i: (i, 0)),
            pl.BlockSpec((BM, D), lambda i: (i, 0)),
            pl.BlockSpec((BM, D), lambda i: (i, 0)),
            pl.BlockSpec((D, D), lambda i: (0, 0)),
            pl.BlockSpec((D, D), lambda i: (0, 0)),
            pl.BlockSpec((D, D), lambda i: (0, 0)),
            pl.BlockSpec((1, OUT), lambda i: (0, 0)),
        ],
        out_specs=pl.BlockSpec((BM, OUT), lambda i: (i, 0)),
        out_shape=jax.ShapeDtypeStruct((BP, OUT), jnp.float32),
    )(selfs, adjm, dism, wt_self, wt_adj, wt_dis, bias2d)


def kernel(nodes, adj_neighbors, dis_neighbors, feat_table,
           W_self, W_adj, W_dis, bias):
    b = nodes.shape[0]
    pad = BP - b
    nodes_p = jnp.concatenate([nodes, jnp.zeros((pad,), jnp.int32)])
    adj_p = jnp.concatenate(
        [adj_neighbors, jnp.zeros((pad, K), jnp.int32)]).reshape(BP * K // 128, 128)
    dis_p = jnp.concatenate(
        [dis_neighbors, jnp.zeros((pad, K), jnp.int32)]).reshape(BP * K // 128, 128)

    selfs, adjm, dism = _sc_gather_mean(nodes_p, adj_p, dis_p, feat_table)
    out = _tc_finish(selfs, adjm, dism, W_self.T, W_adj.T, W_dis.T,
                     bias.reshape(1, OUT))
    return out[:b]


# speedup vs baseline: 1.2924x; 1.2924x over previous
import functools

import jax
import jax.numpy as jnp
from jax import lax
from jax.experimental import pallas as pl
from jax.experimental.pallas import tpu as pltpu
from jax.experimental.pallas import tpu_sc as plsc

D = 128
K = 32
OUT = 384
NC = 2
NS = 16
NW = NC * NS
BP = 10240
CB = BP // NW
C = 8
NCHUNK = CB // C
CK = C * K


def _sc_gather_mean(nodes_p, adj2d, dis2d, feat_table):
    mesh = plsc.VectorSubcoreMesh(core_axis_name="c", subcore_axis_name="s",
                                  num_cores=NC, num_subcores=NS)

    @functools.partial(
        pl.kernel,
        out_type=(
            jax.ShapeDtypeStruct((BP, D), jnp.float32),
            jax.ShapeDtypeStruct((BP, D), jnp.float32),
            jax.ShapeDtypeStruct((BP, D), jnp.float32),
        ),
        mesh=mesh,
        scratch_types=[
            pltpu.VMEM((C,), jnp.int32),
            pltpu.VMEM((2, 128), jnp.int32),
            pltpu.VMEM((2, 128), jnp.int32),
            pltpu.VMEM((C, D), jnp.float32),
            pltpu.VMEM((CK, D), jnp.float32),
            pltpu.VMEM((CK, D), jnp.float32),
            pltpu.VMEM((C, D), jnp.float32),
            pltpu.VMEM((C, D), jnp.float32),
            pltpu.SemaphoreType.DMA,
        ],
    )
    def sc_kernel(nodes_hbm, adj_hbm, dis_hbm, table_hbm,
                  self_out, adj_out, dis_out,
                  nidx, aidx, didx, srows, arows, drows, amean, dmean, sem):
        wid = lax.axis_index("s") * NC + lax.axis_index("c")

        def chunk_body(c, carry):
            base = wid * CB + c * C
            rbase = wid * (CB * K // 128) + c * (C * K // 128)

            pltpu.sync_copy(nodes_hbm.at[pl.ds(base, C)], nidx)
            pltpu.sync_copy(adj_hbm.at[pl.ds(rbase, 2)], aidx)
            pltpu.sync_copy(dis_hbm.at[pl.ds(rbase, 2)], didx)

            cps = [
                pltpu.async_copy(table_hbm.at[nidx], srows, sem),
                pltpu.async_copy(table_hbm.at[aidx.at[0]],
                                 arows.at[pl.ds(0, 128)], sem),
                pltpu.async_copy(table_hbm.at[aidx.at[1]],
                                 arows.at[pl.ds(128, 128)], sem),
                pltpu.async_copy(table_hbm.at[didx.at[0]],
                                 drows.at[pl.ds(0, 128)], sem),
                pltpu.async_copy(table_hbm.at[didx.at[1]],
                                 drows.at[pl.ds(128, 128)], sem),
            ]
            for cp in cps:
                cp.wait()

            def node_body(i, carry2):
                rb = i * K
                for d in range(D // 16):
                    sl = pl.ds(d * 16, 16)

                    def kacc(k4, accs):
                        a, dd = accs
                        for u in range(4):
                            r = rb + k4 * 4 + u
                            a = a + arows[r, sl]
                            dd = dd + drows[r, sl]
                        return (a, dd)

                    zero = jnp.zeros((16,), jnp.float32)
                    a, dd = lax.fori_loop(0, K // 4, kacc, (zero, zero))
                    amean[i, sl] = a * (1.0 / K)
                    dmean[i, sl] = dd * (1.0 / K)
                return carry2

            lax.fori_loop(0, C, node_body, 0)

            pltpu.sync_copy(srows, self_out.at[pl.ds(base, C)])
            pltpu.sync_copy(amean, adj_out.at[pl.ds(base, C)])
            pltpu.sync_copy(dmean, dis_out.at[pl.ds(base, C)])
            return carry

        lax.fori_loop(0, NCHUNK, chunk_body, 0)

    return sc_kernel(nodes_p, adj2d, dis2d, feat_table)


def _tc_finish(selfs, adjm, dism, wt_self, wt_adj, wt_dis, bias2d):
    BM = 512

    def body(s_ref, a_ref, d_ref, ws_ref, wa_ref, wd_ref, b_ref, o_ref):
        hs = jnp.dot(s_ref[...], ws_ref[...], preferred_element_type=jnp.float32)
        ha = jnp.dot(a_ref[...], wa_ref[...], preferred_element_type=jnp.float32)
        hd = jnp.dot(d_ref[...], wd_ref[...], preferred_element_type=jnp.float32)
        h = jnp.concatenate([hs, ha, hd], axis=-1) + b_ref[...]
        h = jnp.where(h >= 0, h, 0.2 * h)
        n = jnp.sqrt(jnp.sum(h * h, axis=-1, keepdims=True))
        o_ref[...] = h / jnp.maximum(n, 1e-12)

    return pl.pallas_call(
        body,
        grid=(BP // BM,),
        in_specs=[
            pl.BlockSpec((BM, D), lambda i: (i, 0)),
            pl.BlockSpec((BM, D), lambda i: (i, 0)),
            pl.BlockSpec((BM, D), lambda i: (i, 0)),
            pl.BlockSpec((D, D), lambda i: (0, 0)),
            pl.BlockSpec((D, D), lambda i: (0, 0)),
            pl.BlockSpec((D, D), lambda i: (0, 0)),
            pl.BlockSpec((1, OUT), lambda i: (0, 0)),
        ],
        out_specs=pl.BlockSpec((BM, OUT), lambda i: (i, 0)),
        out_shape=jax.ShapeDtypeStruct((BP, OUT), jnp.float32),
    )(selfs, adjm, dism, wt_self, wt_adj, wt_dis, bias2d)


def kernel(nodes, adj_neighbors, dis_neighbors, feat_table,
           W_self, W_adj, W_dis, bias):
    b = nodes.shape[0]
    pad = BP - b
    nodes_p = jnp.concatenate([nodes, jnp.zeros((pad,), jnp.int32)])
    adj_p = jnp.concatenate(
        [adj_neighbors, jnp.zeros((pad, K), jnp.int32)]).reshape(BP * K // 128, 128)
    dis_p = jnp.concatenate(
        [dis_neighbors, jnp.zeros((pad, K), jnp.int32)]).reshape(BP * K // 128, 128)

    selfs, adjm, dism = _sc_gather_mean(nodes_p, adj_p, dis_p, feat_table)
    out = _tc_finish(selfs, adjm, dism, W_self.T, W_adj.T, W_dis.T,
                     bias.reshape(1, OUT))
    return out[:b]


# speedup vs baseline: 3.0643x; 2.3710x over previous
import functools

import numpy as np
import jax
import jax.numpy as jnp
from jax import lax
from jax.experimental import pallas as pl
from jax.experimental.pallas import tpu as pltpu
from jax.experimental.pallas import tpu_sc as plsc

D = 128
DW = D // 2
K = 32
OUT = 384
NC = 2
NS = 16
NW = NC * NS
BP = 10240
CB = BP // NW
C = 8
NCHUNK = CB // C
CK = C * K
RPW = CB * K // 128

_p = np.arange(OUT // 3)
_j, _o = _p // 32, _p % 32
PERM = np.where(_o < 16, 32 * _j + 2 * _o, 32 * _j + 2 * (_o - 16) + 1)


def _sc_gather_mean(nodes_p, adj2d, dis2d, table_i32):
    mesh = plsc.VectorSubcoreMesh(core_axis_name="c", subcore_axis_name="s",
                                  num_cores=NC, num_subcores=NS)

    @functools.partial(
        pl.kernel,
        out_type=(
            jax.ShapeDtypeStruct((BP, DW), jnp.int32),
            jax.ShapeDtypeStruct((BP, D), jnp.float32),
            jax.ShapeDtypeStruct((BP, D), jnp.float32),
        ),
        mesh=mesh,
        compiler_params=pltpu.CompilerParams(use_tc_tiling_on_sc=False),
        scratch_types=[
            pltpu.VMEM((CB,), jnp.int32),
            pltpu.VMEM((RPW, 128), jnp.int32),
            pltpu.VMEM((RPW, 128), jnp.int32),
            [pltpu.VMEM((C, DW), jnp.int32)] * 2,
            [pltpu.VMEM((CK, DW), jnp.int32)] * 2,
            [pltpu.VMEM((CK, DW), jnp.int32)] * 2,
            [pltpu.VMEM((C, D), jnp.float32)] * 2,
            [pltpu.VMEM((C, D), jnp.float32)] * 2,
            [pltpu.SemaphoreType.DMA] * 2,
            [pltpu.SemaphoreType.DMA] * 2,
        ],
    )
    def sc_kernel(nodes_hbm, adj_hbm, dis_hbm, tbl,
                  self_out, adj_out, dis_out,
                  nidx, aidx, didx, srows, arows, drows, amean, dmean,
                  gsem, wsem):
        wid = lax.axis_index("s") * NC + lax.axis_index("c")
        pltpu.sync_copy(nodes_hbm.at[pl.ds(wid * CB, CB)], nidx)
        pltpu.sync_copy(adj_hbm.at[pl.ds(wid * RPW, RPW)], aidx)
        pltpu.sync_copy(dis_hbm.at[pl.ds(wid * RPW, RPW)], didx)

        def gather_cps(c, s, make_only):
            mk = pltpu.make_async_copy if make_only else pltpu.async_copy
            return [
                mk(tbl.at[nidx.at[pl.ds(c * C, C)]], srows[s], gsem[s]),
                mk(tbl.at[aidx.at[2 * c]], arows[s].at[pl.ds(0, 128)], gsem[s]),
                mk(tbl.at[aidx.at[2 * c + 1]], arows[s].at[pl.ds(128, 128)], gsem[s]),
                mk(tbl.at[didx.at[2 * c]], drows[s].at[pl.ds(0, 128)], gsem[s]),
                mk(tbl.at[didx.at[2 * c + 1]], drows[s].at[pl.ds(128, 128)], gsem[s]),
            ]

        def wb_cps(base, s, make_only):
            mk = pltpu.make_async_copy if make_only else pltpu.async_copy
            return [
                mk(srows[s], self_out.at[pl.ds(base, C)], wsem[s]),
                mk(amean[s], adj_out.at[pl.ds(base, C)], wsem[s]),
                mk(dmean[s], dis_out.at[pl.ds(base, C)], wsem[s]),
            ]

        gather_cps(0, 0, False)

        inv_k = jnp.full((16,), 1.0 / K, jnp.float32)

        def pair_body(t, carry):
            for s in range(2):
                c = 2 * t + s
                base = wid * CB + c * C

                @pl.when(c + 1 < NCHUNK)
                def _():
                    gather_cps(c + 1, 1 - s, False)

                for cp in gather_cps(c, s, True):
                    cp.wait()

                @pl.when(c >= 2)
                def _():
                    for cp in wb_cps(base, s, True):
                        cp.wait()

                ar, dr = arows[s], drows[s]
                am, dm = amean[s], dmean[s]
                himask = jnp.full((16,), -65536, jnp.int32)

                def node_body(i, carry2):
                    rb = i * K
                    for d in range(4):
                        sl = pl.ds(d * 16, 16)

                        def kacc(k8, accs):
                            aa, ab, da, db = accs
                            for u in range(4):
                                r = rb + k8 * 4 + u
                                wa = ar[r, sl]
                                wd = dr[r, sl]
                                aa = aa + lax.bitcast_convert_type(wa << 16, jnp.float32)
                                ab = ab + lax.bitcast_convert_type(wa & himask, jnp.float32)
                                da = da + lax.bitcast_convert_type(wd << 16, jnp.float32)
                                db = db + lax.bitcast_convert_type(wd & himask, jnp.float32)
                            return (aa, ab, da, db)

                        zero = jnp.zeros((16,), jnp.float32)
                        aa, ab, da, db = lax.fori_loop(
                            0, K // 4, kacc, (zero, zero, zero, zero))
                        am[i, pl.ds(d * 32, 16)] = aa * inv_k
                        am[i, pl.ds(d * 32 + 16, 16)] = ab * inv_k
                        dm[i, pl.ds(d * 32, 16)] = da * inv_k
                        dm[i, pl.ds(d * 32 + 16, 16)] = db * inv_k
                    return carry2

                lax.fori_loop(0, C, node_body, 0)
                wb_cps(base, s, False)
            return carry

        lax.fori_loop(0, NCHUNK // 2, pair_body, 0)

        for s in range(2):
            base = wid * CB + (NCHUNK - 2 + s) * C
            for cp in wb_cps(base, s, True):
                cp.wait()

    return sc_kernel(nodes_p, adj2d, dis2d, table_i32)


def _tc_finish(selfs_bf, adjm, dism, wt_self, wt_adj_p, wt_dis_p, bias2d):
    BM = 512

    def body(s_ref, a_ref, d_ref, ws_ref, wa_ref, wd_ref, b_ref, o_ref):
        s = s_ref[...].astype(jnp.float32)
        hs = jnp.dot(s, ws_ref[...], preferred_element_type=jnp.float32)
        ha = jnp.dot(a_ref[...], wa_ref[...], preferred_element_type=jnp.float32)
        hd = jnp.dot(d_ref[...], wd_ref[...], preferred_element_type=jnp.float32)
        h = jnp.concatenate([hs, ha, hd], axis=-1) + b_ref[...]
        h = jnp.where(h >= 0, h, 0.2 * h)
        n = jnp.sqrt(jnp.sum(h * h, axis=-1, keepdims=True))
        o_ref[...] = h / jnp.maximum(n, 1e-12)

    return pl.pallas_call(
        body,
        grid=(BP // BM,),
        in_specs=[
            pl.BlockSpec((BM, D), lambda i: (i, 0)),
            pl.BlockSpec((BM, D), lambda i: (i, 0)),
            pl.BlockSpec((BM, D), lambda i: (i, 0)),
            pl.BlockSpec((D, D), lambda i: (0, 0)),
            pl.BlockSpec((D, D), lambda i: (0, 0)),
            pl.BlockSpec((D, D), lambda i: (0, 0)),
            pl.BlockSpec((1, OUT), lambda i: (0, 0)),
        ],
        out_specs=pl.BlockSpec((BM, OUT), lambda i: (i, 0)),
        out_shape=jax.ShapeDtypeStruct((BP, OUT), jnp.float32),
    )(selfs_bf, adjm, dism, wt_self, wt_adj_p, wt_dis_p, bias2d)


def kernel(nodes, adj_neighbors, dis_neighbors, feat_table,
           W_self, W_adj, W_dis, bias):
    b = nodes.shape[0]
    pad = BP - b
    nodes_p = jnp.concatenate([nodes, jnp.zeros((pad,), jnp.int32)])
    adj_p = jnp.concatenate(
        [adj_neighbors, jnp.zeros((pad, K), jnp.int32)]).reshape(BP * K // 128, 128)
    dis_p = jnp.concatenate(
        [dis_neighbors, jnp.zeros((pad, K), jnp.int32)]).reshape(BP * K // 128, 128)

    tbl_bf = feat_table.astype(jnp.bfloat16)
    tbl_i32 = lax.bitcast_convert_type(tbl_bf.reshape(-1, DW, 2), jnp.int32)

    selfw, adjm, dism = _sc_gather_mean(nodes_p, adj_p, dis_p, tbl_i32)
    selfs_bf = lax.bitcast_convert_type(selfw, jnp.bfloat16).reshape(BP, D)

    out = _tc_finish(selfs_bf, adjm, dism,
                     W_self.T, W_adj.T[PERM], W_dis.T[PERM],
                     bias.reshape(1, OUT))
    return out[:b]
